# baseline (speedup 1.0000x reference)
import functools
import jax
import jax.numpy as jnp
from jax import lax
from jax.experimental import pallas as pl
from jax.experimental.pallas import tpu as pltpu
from jax.experimental.pallas import tpu_sc as plsc

_NC = 2
_NS = 16
_NW = _NC * _NS
_L = 16

_NROWS = 1000000
_DIM = 32
_NTC_FULL = _NROWS // 128
_TAIL = _NROWS - _NTC_FULL * 128
_SC4_ROWS = 32 * (_NTC_FULL + 1)

_TCB = 4
_TBLK = _NTC_FULL // _TCB
_TPW = _TBLK // _NW

_B = 16384
_F = 26
_BLK = 256
_BPW = _B // _NW
_CPW = _BPW // _BLK
_NBLK = _F * _CPW


def _transpose_block(staged, tbuf, s, ncols):
    iota = lax.iota(jnp.int32, _L)
    rowp = iota >> 2
    colp = (iota & 3) * _DIM
    cols_j = [colp + j for j in range(_DIM)]

    @plsc.parallel_loop(0, ncols // _L, unroll=4)
    def do_g(g):
        rows = rowp + 4 * g
        for j in range(_DIM):
            val = staged[s, j, pl.ds(g * _L, _L)]
            plsc.store_scatter(tbuf.at[s], [rows, cols_j[j]], val)


def _transpose_body(tab_hbm, sc4_hbm, staged, tail_st, tbuf,
                    sem_i0, sem_i1, sem_o0, sem_o1):
    sem_i = (sem_i0, sem_i1)
    sem_o = (sem_o0, sem_o1)
    w = lax.axis_index("s") * _NC + lax.axis_index("c")
    blk0 = w * _TPW

    def lane0(b):
        return (blk0 + b) * _TCB * 128

    def start_in(b, s):
        pltpu.async_copy(tab_hbm.at[:, pl.ds(lane0(b), _TCB * 128)],
                         staged.at[s], sem_i[s])

    def wait_in(b, s):
        pltpu.make_async_copy(tab_hbm.at[:, pl.ds(lane0(b), _TCB * 128)],
                              staged.at[s], sem_i[s]).wait()

    def out_dst(b):
        return sc4_hbm.at[pl.ds((blk0 + b) * _TCB * 32, _TCB * 32), :]

    def start_out(b, s):
        pltpu.async_copy(tbuf.at[s], out_dst(b), sem_o[s])

    def wait_out(b, s):
        pltpu.make_async_copy(tbuf.at[s], out_dst(b), sem_o[s]).wait()

    start_in(0, 0)
    start_in(1, 1)

    def pair(p, carry):
        ba = 2 * p
        bb = ba + 1
        wait_in(ba, 0)

        @pl.when(p >= 1)
        def _wo0():
            wait_out(ba - 2, 0)
        _transpose_block(staged, tbuf, 0, _TCB * 128)
        start_out(ba, 0)
        start_in(ba + 2, 0)

        wait_in(bb, 1)

        @pl.when(p >= 1)
        def _wo1():
            wait_out(bb - 2, 1)
        _transpose_block(staged, tbuf, 1, _TCB * 128)
        start_out(bb, 1)

        @pl.when(p < _TPW // 2 - 1)
        def _gi1():
            start_in(bb + 2, 1)
        return carry

    lax.fori_loop(0, _TPW // 2, pair, 0)
    b_last = _TPW - 1
    wait_in(b_last, 0)
    wait_out(b_last - 2, 0)
    _transpose_block(staged, tbuf, 0, _TCB * 128)
    start_out(b_last, 0)
    wait_out(b_last - 1, 1)
    wait_out(b_last, 0)

    @pl.when(w == _NW - 1)
    def _extra():
        pltpu.sync_copy(tab_hbm.at[:, pl.ds(1952 * _TCB * 128, _TCB * 128)],
                        staged.at[0])
        _transpose_block(staged, tbuf, 0, _TCB * 128)
        pltpu.sync_copy(tbuf.at[0],
                        sc4_hbm.at[pl.ds(1952 * _TCB * 32, _TCB * 32), :])

        pltpu.sync_copy(tab_hbm.at[:, pl.ds(_NTC_FULL * 128, _TAIL)], tail_st)
        iota = lax.iota(jnp.int32, _L)
        rowp = iota >> 2
        colp = (iota & 3) * _DIM

        @plsc.parallel_loop(0, _TAIL // _L, unroll=2)
        def do_g(g):
            rows = rowp + 4 * g
            for j in range(_DIM):
                val = tail_st[j, pl.ds(g * _L, _L)]
                plsc.store_scatter(tbuf.at[0], [rows, colp + j], val)
        pltpu.sync_copy(tbuf.at[0, pl.ds(0, _TAIL // 4)],
                        sc4_hbm.at[pl.ds(_NTC_FULL * 32, _TAIL // 4), :])


def _gather_body(ind_hbm, sc4_hbm, out_hbm, idxst, qbuf, gbuf, ost,
                 sem_g0, sem_g1, sem_o0, sem_o1):
    sem_g = (sem_g0, sem_g1)
    sem_o = (sem_o0, sem_o1)
    w = lax.axis_index("s") * _NC + lax.axis_index("c")
    b0w = w * _BPW
    iota = lax.iota(jnp.int32, _L)

    pltpu.sync_copy(ind_hbm.at[:, pl.ds(b0w, _BPW)], idxst)

    def start_gather(blk, s):
        f = blk // _CPW
        c2 = blk % _CPW

        @plsc.parallel_loop(0, _BLK // _L, unroll=2)
        def mk_q(g):
            v = idxst[f, pl.ds(c2 * _BLK + g * _L, _L)]
            qbuf[s, g // 8, pl.ds((g % 8) * _L, _L)] = v >> 2
        for h in range(_BLK // 128):
            pltpu.async_copy(sc4_hbm.at[qbuf.at[s, h]],
                             gbuf.at[s, pl.ds(h * 128, 128), :], sem_g[s])

    def wait_gather(s):
        for h in range(_BLK // 128):
            pltpu.make_async_copy(sc4_hbm.at[qbuf.at[s, h]],
                                  gbuf.at[s, pl.ds(h * 128, 128), :],
                                  sem_g[s]).wait()

    def out_ref(blk):
        f = blk // _CPW
        c2 = blk % _CPW
        return out_hbm.at[f, :, pl.ds(b0w + c2 * _BLK, _BLK)]

    def select_and_write(blk, s):
        f = blk // _CPW
        c2 = blk % _CPW

        @plsc.parallel_loop(0, _BLK // _L, unroll=4)
        def do_g(g):
            db = iota + g * _L
            k = idxst[f, pl.ds(c2 * _BLK + g * _L, _L)] & 3
            colbase = k * _DIM
            for j in range(_DIM):
                val = plsc.load_gather(gbuf.at[s], [db, colbase + j])
                ost[s, j, pl.ds(g * _L, _L)] = val
        pltpu.async_copy(ost.at[s], out_ref(blk), sem_o[s])

    def wait_out(blk, s):
        pltpu.make_async_copy(ost.at[s], out_ref(blk), sem_o[s]).wait()

    start_gather(0, 0)
    start_gather(1, 1)

    def pair(p, carry):
        blk_a = 2 * p
        blk_b = blk_a + 1

        @pl.when(p >= 1)
        def _wo0():
            wait_out(blk_a - 2, 0)
        wait_gather(0)
        select_and_write(blk_a, 0)

        @pl.when(p < _NBLK // 2 - 1)
        def _g0():
            start_gather(blk_a + 2, 0)

        @pl.when(p >= 1)
        def _wo1():
            wait_out(blk_b - 2, 1)
        wait_gather(1)
        select_and_write(blk_b, 1)

        @pl.when(p < _NBLK // 2 - 1)
        def _g1():
            start_gather(blk_b + 2, 1)
        return carry

    lax.fori_loop(0, _NBLK // 2, pair, 0)
    wait_out(_NBLK - 2, 0)
    wait_out(_NBLK - 1, 1)


def kernel(indic, table):
    ind_t = indic.T
    tab_t = table.T

    transpose = functools.partial(
        pl.kernel,
        out_type=jax.ShapeDtypeStruct((_SC4_ROWS, 128), jnp.float32),
        scratch_types=[
            pltpu.VMEM((2, _DIM, _TCB * 128), jnp.float32),
            pltpu.VMEM((_DIM, _TAIL), jnp.float32),
            pltpu.VMEM((2, _TCB * 32, 128), jnp.float32),
        ] + [pltpu.SemaphoreType.DMA] * 4,
        mesh=plsc.VectorSubcoreMesh(core_axis_name="c", subcore_axis_name="s"),
        compiler_params=pltpu.CompilerParams(use_tc_tiling_on_sc=True, needs_layout_passes=False),
    )(_transpose_body)

    gather = functools.partial(
        pl.kernel,
        out_type=jax.ShapeDtypeStruct((_F, _DIM, _B), jnp.float32),
        scratch_types=[
            pltpu.VMEM((_F, _BPW), jnp.int32),
            pltpu.VMEM((2, _BLK // 128, 128), jnp.int32),
            pltpu.VMEM((2, _BLK, 128), jnp.float32),
            pltpu.VMEM((2, _DIM, _BLK), jnp.float32),
        ] + [pltpu.SemaphoreType.DMA] * 4,
        mesh=plsc.VectorSubcoreMesh(core_axis_name="c", subcore_axis_name="s"),
        compiler_params=pltpu.CompilerParams(use_tc_tiling_on_sc=True, needs_layout_passes=False),
    )(_gather_body)

    sc4 = transpose(tab_t)
    out_t = gather(ind_t, sc4)
    return out_t.transpose(2, 0, 1)


# speedup vs baseline: 1.0170x; 1.0170x over previous
import functools
import jax
import jax.numpy as jnp
from jax import lax
from jax.experimental import pallas as pl
from jax.experimental.pallas import tpu as pltpu
from jax.experimental.pallas import tpu_sc as plsc

_NC = 2
_NS = 16
_NW = _NC * _NS
_L = 16

_NROWS = 1000000
_DIM = 32
_NTC_FULL = _NROWS // 128
_TAIL = _NROWS - _NTC_FULL * 128
_SC4_ROWS = 32 * (_NTC_FULL + 1)

_TCB = 4
_TBLK = _NTC_FULL // _TCB
_TPW = _TBLK // _NW

_B = 16384
_F = 26
_BLK = 256
_BPW = _B // _NW
_CPW = _BPW // _BLK
_NBLK = _F * _CPW


def _transpose_block(staged, tbuf, s, ncols):
    iota = lax.iota(jnp.int32, _L)
    rowp = iota >> 2
    colp = (iota & 3) * _DIM
    cols_j = [colp + j for j in range(_DIM)]

    @plsc.parallel_loop(0, ncols // _L, unroll=2)
    def do_g(g):
        rows = rowp + 4 * g
        for j in range(_DIM):
            val = staged[s, j, pl.ds(g * _L, _L)]
            plsc.store_scatter(tbuf.at[s], [rows, cols_j[j]], val)


def _transpose_body(tab_hbm, sc4_hbm, staged, tail_st, tbuf,
                    sem_i0, sem_i1, sem_o0, sem_o1):
    sem_i = (sem_i0, sem_i1)
    sem_o = (sem_o0, sem_o1)
    w = lax.axis_index("s") * _NC + lax.axis_index("c")
    blk0 = w * _TPW

    def lane0(b):
        return (blk0 + b) * _TCB * 128

    def start_in(b, s):
        pltpu.async_copy(tab_hbm.at[:, pl.ds(lane0(b), _TCB * 128)],
                         staged.at[s], sem_i[s])

    def wait_in(b, s):
        pltpu.make_async_copy(tab_hbm.at[:, pl.ds(lane0(b), _TCB * 128)],
                              staged.at[s], sem_i[s]).wait()

    def out_dst(b):
        return sc4_hbm.at[pl.ds((blk0 + b) * _TCB * 32, _TCB * 32), :]

    def start_out(b, s):
        pltpu.async_copy(tbuf.at[s], out_dst(b), sem_o[s])

    def wait_out(b, s):
        pltpu.make_async_copy(tbuf.at[s], out_dst(b), sem_o[s]).wait()

    start_in(0, 0)
    start_in(1, 1)

    def pair(p, carry):
        ba = 2 * p
        bb = ba + 1
        wait_in(ba, 0)

        @pl.when(p >= 1)
        def _wo0():
            wait_out(ba - 2, 0)
        _transpose_block(staged, tbuf, 0, _TCB * 128)
        start_out(ba, 0)
        start_in(ba + 2, 0)

        wait_in(bb, 1)

        @pl.when(p >= 1)
        def _wo1():
            wait_out(bb - 2, 1)
        _transpose_block(staged, tbuf, 1, _TCB * 128)
        start_out(bb, 1)

        @pl.when(p < _TPW // 2 - 1)
        def _gi1():
            start_in(bb + 2, 1)
        return carry

    lax.fori_loop(0, _TPW // 2, pair, 0)
    b_last = _TPW - 1
    wait_in(b_last, 0)
    wait_out(b_last - 2, 0)
    _transpose_block(staged, tbuf, 0, _TCB * 128)
    start_out(b_last, 0)
    wait_out(b_last - 1, 1)
    wait_out(b_last, 0)

    @pl.when(w == _NW - 1)
    def _extra():
        pltpu.sync_copy(tab_hbm.at[:, pl.ds(1952 * _TCB * 128, _TCB * 128)],
                        staged.at[0])
        _transpose_block(staged, tbuf, 0, _TCB * 128)
        pltpu.sync_copy(tbuf.at[0],
                        sc4_hbm.at[pl.ds(1952 * _TCB * 32, _TCB * 32), :])

        pltpu.sync_copy(tab_hbm.at[:, pl.ds(_NTC_FULL * 128, _TAIL)], tail_st)
        iota = lax.iota(jnp.int32, _L)
        rowp = iota >> 2
        colp = (iota & 3) * _DIM

        @plsc.parallel_loop(0, _TAIL // _L, unroll=2)
        def do_g(g):
            rows = rowp + 4 * g
            for j in range(_DIM):
                val = tail_st[j, pl.ds(g * _L, _L)]
                plsc.store_scatter(tbuf.at[0], [rows, colp + j], val)
        pltpu.sync_copy(tbuf.at[0, pl.ds(0, _TAIL // 4)],
                        sc4_hbm.at[pl.ds(_NTC_FULL * 32, _TAIL // 4), :])


def _gather_body(ind_hbm, sc4_hbm, out_hbm, idxst, qbuf, gbuf, ost,
                 sem_g0, sem_g1, sem_o0, sem_o1):
    sem_g = (sem_g0, sem_g1)
    sem_o = (sem_o0, sem_o1)
    w = lax.axis_index("s") * _NC + lax.axis_index("c")
    b0w = w * _BPW
    iota = lax.iota(jnp.int32, _L)

    pltpu.sync_copy(ind_hbm.at[:, pl.ds(b0w, _BPW)], idxst)

    def start_gather(blk, s):
        f = blk // _CPW
        c2 = blk % _CPW

        @plsc.parallel_loop(0, _BLK // _L, unroll=2)
        def mk_q(g):
            v = idxst[f, pl.ds(c2 * _BLK + g * _L, _L)]
            qbuf[s, g // 8, pl.ds((g % 8) * _L, _L)] = v >> 2
        for h in range(_BLK // 128):
            pltpu.async_copy(sc4_hbm.at[qbuf.at[s, h]],
                             gbuf.at[s, pl.ds(h * 128, 128), :], sem_g[s])

    def wait_gather(s):
        for h in range(_BLK // 128):
            pltpu.make_async_copy(sc4_hbm.at[qbuf.at[s, h]],
                                  gbuf.at[s, pl.ds(h * 128, 128), :],
                                  sem_g[s]).wait()

    def out_ref(blk):
        f = blk // _CPW
        c2 = blk % _CPW
        return out_hbm.at[f, :, pl.ds(b0w + c2 * _BLK, _BLK)]

    def select_and_write(blk, s):
        f = blk // _CPW
        c2 = blk % _CPW

        @plsc.parallel_loop(0, _BLK // _L, unroll=2)
        def do_g(g):
            db = iota + g * _L
            k = idxst[f, pl.ds(c2 * _BLK + g * _L, _L)] & 3
            colbase = k * _DIM
            for j in range(_DIM):
                val = plsc.load_gather(gbuf.at[s], [db, colbase + j])
                ost[s, j, pl.ds(g * _L, _L)] = val
        pltpu.async_copy(ost.at[s], out_ref(blk), sem_o[s])

    def wait_out(blk, s):
        pltpu.make_async_copy(ost.at[s], out_ref(blk), sem_o[s]).wait()

    start_gather(0, 0)
    start_gather(1, 1)

    def pair(p, carry):
        blk_a = 2 * p
        blk_b = blk_a + 1

        @pl.when(p >= 1)
        def _wo0():
            wait_out(blk_a - 2, 0)
        wait_gather(0)
        select_and_write(blk_a, 0)

        @pl.when(p < _NBLK // 2 - 1)
        def _g0():
            start_gather(blk_a + 2, 0)

        @pl.when(p >= 1)
        def _wo1():
            wait_out(blk_b - 2, 1)
        wait_gather(1)
        select_and_write(blk_b, 1)

        @pl.when(p < _NBLK // 2 - 1)
        def _g1():
            start_gather(blk_b + 2, 1)
        return carry

    lax.fori_loop(0, _NBLK // 2, pair, 0)
    wait_out(_NBLK - 2, 0)
    wait_out(_NBLK - 1, 1)


def kernel(indic, table):
    ind_t = indic.T
    tab_t = table.T

    transpose = functools.partial(
        pl.kernel,
        out_type=jax.ShapeDtypeStruct((_SC4_ROWS, 128), jnp.float32),
        scratch_types=[
            pltpu.VMEM((2, _DIM, _TCB * 128), jnp.float32),
            pltpu.VMEM((_DIM, _TAIL), jnp.float32),
            pltpu.VMEM((2, _TCB * 32, 128), jnp.float32),
        ] + [pltpu.SemaphoreType.DMA] * 4,
        mesh=plsc.VectorSubcoreMesh(core_axis_name="c", subcore_axis_name="s"),
        compiler_params=pltpu.CompilerParams(use_tc_tiling_on_sc=True, needs_layout_passes=False),
    )(_transpose_body)

    gather = functools.partial(
        pl.kernel,
        out_type=jax.ShapeDtypeStruct((_F, _DIM, _B), jnp.float32),
        scratch_types=[
            pltpu.VMEM((_F, _BPW), jnp.int32),
            pltpu.VMEM((2, _BLK // 128, 128), jnp.int32),
            pltpu.VMEM((2, _BLK, 128), jnp.float32),
            pltpu.VMEM((2, _DIM, _BLK), jnp.float32),
        ] + [pltpu.SemaphoreType.DMA] * 4,
        mesh=plsc.VectorSubcoreMesh(core_axis_name="c", subcore_axis_name="s"),
        compiler_params=pltpu.CompilerParams(use_tc_tiling_on_sc=True, needs_layout_passes=False),
    )(_gather_body)

    sc4 = transpose(tab_t)
    out_t = gather(ind_t, sc4)
    return out_t.transpose(2, 0, 1)


# speedup vs baseline: 1.1127x; 1.0941x over previous
import functools
import jax
import jax.numpy as jnp
from jax import lax
from jax.experimental import pallas as pl
from jax.experimental.pallas import tpu as pltpu
from jax.experimental.pallas import tpu_sc as plsc

_NC = 2
_NS = 16
_NW = _NC * _NS
_L = 16

_NROWS = 1000000
_DIM = 32
_NTC_FULL = _NROWS // 128
_TAIL = _NROWS - _NTC_FULL * 128
_SC4_ROWS = 32 * (_NTC_FULL + 1)

_TCB = 4
_TBLK = _NTC_FULL // _TCB
_TPW = _TBLK // _NW

_B = 16384
_F = 26
_BLK = 256
_BPW = _B // _NW
_CPW = _BPW // _BLK
_NBLK = _F * _CPW


def _transpose_block(staged, tbuf, s, ncols):
    iota = lax.iota(jnp.int32, _L)
    rows01 = (iota, iota + _L)

    @plsc.parallel_loop(0, ncols // 4, unroll=2)
    def do_dq(dq):
        c0 = 4 * dq
        for v in range(8):
            cols = jnp.full((_L,), c0 + (v // 2), jnp.int32)
            val = plsc.load_gather(staged.at[s], [rows01[v % 2], cols])
            tbuf[s, dq, pl.ds(16 * v, 16)] = val


def _transpose_body(tab_hbm, sc4_hbm, staged, tail_st, tbuf,
                    sem_i0, sem_i1, sem_o0, sem_o1):
    sem_i = (sem_i0, sem_i1)
    sem_o = (sem_o0, sem_o1)
    w = lax.axis_index("s") * _NC + lax.axis_index("c")
    blk0 = w * _TPW

    def lane0(b):
        return (blk0 + b) * _TCB * 128

    def start_in(b, s):
        pltpu.async_copy(tab_hbm.at[:, pl.ds(lane0(b), _TCB * 128)],
                         staged.at[s], sem_i[s])

    def wait_in(b, s):
        pltpu.make_async_copy(tab_hbm.at[:, pl.ds(lane0(b), _TCB * 128)],
                              staged.at[s], sem_i[s]).wait()

    def out_dst(b):
        return sc4_hbm.at[pl.ds((blk0 + b) * _TCB * 32, _TCB * 32), :]

    def start_out(b, s):
        pltpu.async_copy(tbuf.at[s], out_dst(b), sem_o[s])

    def wait_out(b, s):
        pltpu.make_async_copy(tbuf.at[s], out_dst(b), sem_o[s]).wait()

    start_in(0, 0)
    start_in(1, 1)

    def pair(p, carry):
        ba = 2 * p
        bb = ba + 1
        wait_in(ba, 0)

        @pl.when(p >= 1)
        def _wo0():
            wait_out(ba - 2, 0)
        _transpose_block(staged, tbuf, 0, _TCB * 128)
        start_out(ba, 0)
        start_in(ba + 2, 0)

        wait_in(bb, 1)

        @pl.when(p >= 1)
        def _wo1():
            wait_out(bb - 2, 1)
        _transpose_block(staged, tbuf, 1, _TCB * 128)
        start_out(bb, 1)

        @pl.when(p < _TPW // 2 - 1)
        def _gi1():
            start_in(bb + 2, 1)
        return carry

    lax.fori_loop(0, _TPW // 2, pair, 0)
    b_last = _TPW - 1
    wait_in(b_last, 0)
    wait_out(b_last - 2, 0)
    _transpose_block(staged, tbuf, 0, _TCB * 128)
    start_out(b_last, 0)
    wait_out(b_last - 1, 1)
    wait_out(b_last, 0)

    @pl.when(w == _NW - 1)
    def _extra():
        pltpu.sync_copy(tab_hbm.at[:, pl.ds(1952 * _TCB * 128, _TCB * 128)],
                        staged.at[0])
        _transpose_block(staged, tbuf, 0, _TCB * 128)
        pltpu.sync_copy(tbuf.at[0],
                        sc4_hbm.at[pl.ds(1952 * _TCB * 32, _TCB * 32), :])

        pltpu.sync_copy(tab_hbm.at[:, pl.ds(_NTC_FULL * 128, _TAIL)], tail_st)
        iota = lax.iota(jnp.int32, _L)
        rowp = iota >> 2
        colp = (iota & 3) * _DIM

        @plsc.parallel_loop(0, _TAIL // _L, unroll=2)
        def do_g(g):
            rows = rowp + 4 * g
            for j in range(_DIM):
                val = tail_st[j, pl.ds(g * _L, _L)]
                plsc.store_scatter(tbuf.at[0], [rows, colp + j], val)
        pltpu.sync_copy(tbuf.at[0, pl.ds(0, _TAIL // 4)],
                        sc4_hbm.at[pl.ds(_NTC_FULL * 32, _TAIL // 4), :])


def _gather_body(ind_hbm, sc4_hbm, out_hbm, idxst, qbuf, gbuf, ost,
                 sem_g0, sem_g1, sem_o0, sem_o1):
    sem_g = (sem_g0, sem_g1)
    sem_o = (sem_o0, sem_o1)
    w = lax.axis_index("s") * _NC + lax.axis_index("c")
    b0w = w * _BPW
    iota = lax.iota(jnp.int32, _L)

    pltpu.sync_copy(ind_hbm.at[:, pl.ds(b0w, _BPW)], idxst)

    def start_gather(blk, s):
        f = blk // _CPW
        c2 = blk % _CPW

        @plsc.parallel_loop(0, _BLK // _L, unroll=2)
        def mk_q(g):
            v = idxst[f, pl.ds(c2 * _BLK + g * _L, _L)]
            qbuf[s, g // 8, pl.ds((g % 8) * _L, _L)] = v >> 2
        for h in range(_BLK // 128):
            pltpu.async_copy(sc4_hbm.at[qbuf.at[s, h]],
                             gbuf.at[s, pl.ds(h * 128, 128), :], sem_g[s])

    def wait_gather(s):
        for h in range(_BLK // 128):
            pltpu.make_async_copy(sc4_hbm.at[qbuf.at[s, h]],
                                  gbuf.at[s, pl.ds(h * 128, 128), :],
                                  sem_g[s]).wait()

    def out_ref(blk):
        f = blk // _CPW
        c2 = blk % _CPW
        return out_hbm.at[f, :, pl.ds(b0w + c2 * _BLK, _BLK)]

    def select_and_write(blk, s):
        f = blk // _CPW
        c2 = blk % _CPW

        @plsc.parallel_loop(0, _BLK // _L, unroll=2)
        def do_g(g):
            db = iota + g * _L
            k = idxst[f, pl.ds(c2 * _BLK + g * _L, _L)] & 3
            colbase = k * _DIM
            for j in range(_DIM):
                val = plsc.load_gather(gbuf.at[s], [db, colbase + j])
                ost[s, j, pl.ds(g * _L, _L)] = val
        pltpu.async_copy(ost.at[s], out_ref(blk), sem_o[s])

    def wait_out(blk, s):
        pltpu.make_async_copy(ost.at[s], out_ref(blk), sem_o[s]).wait()

    start_gather(0, 0)
    start_gather(1, 1)

    def pair(p, carry):
        blk_a = 2 * p
        blk_b = blk_a + 1

        @pl.when(p >= 1)
        def _wo0():
            wait_out(blk_a - 2, 0)
        wait_gather(0)
        select_and_write(blk_a, 0)

        @pl.when(p < _NBLK // 2 - 1)
        def _g0():
            start_gather(blk_a + 2, 0)

        @pl.when(p >= 1)
        def _wo1():
            wait_out(blk_b - 2, 1)
        wait_gather(1)
        select_and_write(blk_b, 1)

        @pl.when(p < _NBLK // 2 - 1)
        def _g1():
            start_gather(blk_b + 2, 1)
        return carry

    lax.fori_loop(0, _NBLK // 2, pair, 0)
    wait_out(_NBLK - 2, 0)
    wait_out(_NBLK - 1, 1)


def kernel(indic, table):
    ind_t = indic.T
    tab_t = table.T

    transpose = functools.partial(
        pl.kernel,
        out_type=jax.ShapeDtypeStruct((_SC4_ROWS, 128), jnp.float32),
        scratch_types=[
            pltpu.VMEM((2, _DIM, _TCB * 128), jnp.float32),
            pltpu.VMEM((_DIM, _TAIL), jnp.float32),
            pltpu.VMEM((2, _TCB * 32, 128), jnp.float32),
        ] + [pltpu.SemaphoreType.DMA] * 4,
        mesh=plsc.VectorSubcoreMesh(core_axis_name="c", subcore_axis_name="s"),
        compiler_params=pltpu.CompilerParams(use_tc_tiling_on_sc=True, needs_layout_passes=False),
    )(_transpose_body)

    gather = functools.partial(
        pl.kernel,
        out_type=jax.ShapeDtypeStruct((_F, _DIM, _B), jnp.float32),
        scratch_types=[
            pltpu.VMEM((_F, _BPW), jnp.int32),
            pltpu.VMEM((2, _BLK // 128, 128), jnp.int32),
            pltpu.VMEM((2, _BLK, 128), jnp.float32),
            pltpu.VMEM((2, _DIM, _BLK), jnp.float32),
        ] + [pltpu.SemaphoreType.DMA] * 4,
        mesh=plsc.VectorSubcoreMesh(core_axis_name="c", subcore_axis_name="s"),
        compiler_params=pltpu.CompilerParams(use_tc_tiling_on_sc=True, needs_layout_passes=False),
    )(_gather_body)

    sc4 = transpose(tab_t)
    out_t = gather(ind_t, sc4)
    return out_t.transpose(2, 0, 1)


# speedup vs baseline: 1.1143x; 1.0014x over previous
import functools
import jax
import jax.numpy as jnp
from jax import lax
from jax.experimental import pallas as pl
from jax.experimental.pallas import tpu as pltpu
from jax.experimental.pallas import tpu_sc as plsc

_NC = 2
_NS = 16
_NW = _NC * _NS
_L = 16

_NROWS = 1000000
_DIM = 32
_NTC_FULL = _NROWS // 128
_TAIL = _NROWS - _NTC_FULL * 128
_SC4_ROWS = 32 * (_NTC_FULL + 1)

_TCB = 4
_TBLK = _NTC_FULL // _TCB
_TPW = _TBLK // _NW

_B = 16384
_F = 26
_BLK = 256
_BPW = _B // _NW
_CPW = _BPW // _BLK
_NBLK = _F * _CPW


def _transpose_block(staged, tbuf, s, ncols):
    iota = lax.iota(jnp.int32, _L)
    rows01 = (iota, iota + _L)

    @plsc.parallel_loop(0, ncols // 4, unroll=4)
    def do_dq(dq):
        c0 = 4 * dq
        for v in range(8):
            cols = jnp.full((_L,), c0 + (v // 2), jnp.int32)
            val = plsc.load_gather(staged.at[s], [rows01[v % 2], cols])
            tbuf[s, dq, pl.ds(16 * v, 16)] = val


def _transpose_body(tab_hbm, sc4_hbm, staged, tail_st, tbuf,
                    sem_i0, sem_i1, sem_o0, sem_o1):
    sem_i = (sem_i0, sem_i1)
    sem_o = (sem_o0, sem_o1)
    w = lax.axis_index("s") * _NC + lax.axis_index("c")
    blk0 = w * _TPW

    def lane0(b):
        return (blk0 + b) * _TCB * 128

    def start_in(b, s):
        pltpu.async_copy(tab_hbm.at[:, pl.ds(lane0(b), _TCB * 128)],
                         staged.at[s], sem_i[s])

    def wait_in(b, s):
        pltpu.make_async_copy(tab_hbm.at[:, pl.ds(lane0(b), _TCB * 128)],
                              staged.at[s], sem_i[s]).wait()

    def out_dst(b):
        return sc4_hbm.at[pl.ds((blk0 + b) * _TCB * 32, _TCB * 32), :]

    def start_out(b, s):
        pltpu.async_copy(tbuf.at[s], out_dst(b), sem_o[s])

    def wait_out(b, s):
        pltpu.make_async_copy(tbuf.at[s], out_dst(b), sem_o[s]).wait()

    start_in(0, 0)
    start_in(1, 1)

    def pair(p, carry):
        ba = 2 * p
        bb = ba + 1
        wait_in(ba, 0)

        @pl.when(p >= 1)
        def _wo0():
            wait_out(ba - 2, 0)
        _transpose_block(staged, tbuf, 0, _TCB * 128)
        start_out(ba, 0)
        start_in(ba + 2, 0)

        wait_in(bb, 1)

        @pl.when(p >= 1)
        def _wo1():
            wait_out(bb - 2, 1)
        _transpose_block(staged, tbuf, 1, _TCB * 128)
        start_out(bb, 1)

        @pl.when(p < _TPW // 2 - 1)
        def _gi1():
            start_in(bb + 2, 1)
        return carry

    lax.fori_loop(0, _TPW // 2, pair, 0)
    b_last = _TPW - 1
    wait_in(b_last, 0)
    wait_out(b_last - 2, 0)
    _transpose_block(staged, tbuf, 0, _TCB * 128)
    start_out(b_last, 0)
    wait_out(b_last - 1, 1)
    wait_out(b_last, 0)

    @pl.when(w == _NW - 1)
    def _extra():
        pltpu.sync_copy(tab_hbm.at[:, pl.ds(1952 * _TCB * 128, _TCB * 128)],
                        staged.at[0])
        _transpose_block(staged, tbuf, 0, _TCB * 128)
        pltpu.sync_copy(tbuf.at[0],
                        sc4_hbm.at[pl.ds(1952 * _TCB * 32, _TCB * 32), :])

        pltpu.sync_copy(tab_hbm.at[:, pl.ds(_NTC_FULL * 128, _TAIL)], tail_st)
        iota = lax.iota(jnp.int32, _L)
        rowp = iota >> 2
        colp = (iota & 3) * _DIM

        @plsc.parallel_loop(0, _TAIL // _L, unroll=2)
        def do_g(g):
            rows = rowp + 4 * g
            for j in range(_DIM):
                val = tail_st[j, pl.ds(g * _L, _L)]
                plsc.store_scatter(tbuf.at[0], [rows, colp + j], val)
        pltpu.sync_copy(tbuf.at[0, pl.ds(0, _TAIL // 4)],
                        sc4_hbm.at[pl.ds(_NTC_FULL * 32, _TAIL // 4), :])


def _gather_body(ind_hbm, sc4_hbm, out_hbm, idxst, qbuf, gbuf, ost,
                 sem_g0, sem_g1, sem_o0, sem_o1):
    sem_g = (sem_g0, sem_g1)
    sem_o = (sem_o0, sem_o1)
    w = lax.axis_index("s") * _NC + lax.axis_index("c")
    b0w = w * _BPW
    iota = lax.iota(jnp.int32, _L)

    pltpu.sync_copy(ind_hbm.at[:, pl.ds(b0w, _BPW)], idxst)

    def start_gather(blk, s):
        f = blk // _CPW
        c2 = blk % _CPW

        @plsc.parallel_loop(0, _BLK // _L, unroll=2)
        def mk_q(g):
            v = idxst[f, pl.ds(c2 * _BLK + g * _L, _L)]
            qbuf[s, g // 8, pl.ds((g % 8) * _L, _L)] = v >> 2
        for h in range(_BLK // 128):
            pltpu.async_copy(sc4_hbm.at[qbuf.at[s, h]],
                             gbuf.at[s, pl.ds(h * 128, 128), :], sem_g[s])

    def wait_gather(s):
        for h in range(_BLK // 128):
            pltpu.make_async_copy(sc4_hbm.at[qbuf.at[s, h]],
                                  gbuf.at[s, pl.ds(h * 128, 128), :],
                                  sem_g[s]).wait()

    def out_ref(blk):
        f = blk // _CPW
        c2 = blk % _CPW
        return out_hbm.at[f, :, pl.ds(b0w + c2 * _BLK, _BLK)]

    def select_and_write(blk, s):
        f = blk // _CPW
        c2 = blk % _CPW

        @plsc.parallel_loop(0, _BLK // _L, unroll=2)
        def do_g(g):
            db = iota + g * _L
            k = idxst[f, pl.ds(c2 * _BLK + g * _L, _L)] & 3
            colbase = k * _DIM
            for j in range(_DIM):
                val = plsc.load_gather(gbuf.at[s], [db, colbase + j])
                ost[s, j, pl.ds(g * _L, _L)] = val
        pltpu.async_copy(ost.at[s], out_ref(blk), sem_o[s])

    def wait_out(blk, s):
        pltpu.make_async_copy(ost.at[s], out_ref(blk), sem_o[s]).wait()

    start_gather(0, 0)
    start_gather(1, 1)

    def pair(p, carry):
        blk_a = 2 * p
        blk_b = blk_a + 1

        @pl.when(p >= 1)
        def _wo0():
            wait_out(blk_a - 2, 0)
        wait_gather(0)
        select_and_write(blk_a, 0)

        @pl.when(p < _NBLK // 2 - 1)
        def _g0():
            start_gather(blk_a + 2, 0)

        @pl.when(p >= 1)
        def _wo1():
            wait_out(blk_b - 2, 1)
        wait_gather(1)
        select_and_write(blk_b, 1)

        @pl.when(p < _NBLK // 2 - 1)
        def _g1():
            start_gather(blk_b + 2, 1)
        return carry

    lax.fori_loop(0, _NBLK // 2, pair, 0)
    wait_out(_NBLK - 2, 0)
    wait_out(_NBLK - 1, 1)


def kernel(indic, table):
    ind_t = indic.T
    tab_t = table.T

    transpose = functools.partial(
        pl.kernel,
        out_type=jax.ShapeDtypeStruct((_SC4_ROWS, 128), jnp.float32),
        scratch_types=[
            pltpu.VMEM((2, _DIM, _TCB * 128), jnp.float32),
            pltpu.VMEM((_DIM, _TAIL), jnp.float32),
            pltpu.VMEM((2, _TCB * 32, 128), jnp.float32),
        ] + [pltpu.SemaphoreType.DMA] * 4,
        mesh=plsc.VectorSubcoreMesh(core_axis_name="c", subcore_axis_name="s"),
        compiler_params=pltpu.CompilerParams(use_tc_tiling_on_sc=True, needs_layout_passes=False),
    )(_transpose_body)

    gather = functools.partial(
        pl.kernel,
        out_type=jax.ShapeDtypeStruct((_F, _DIM, _B), jnp.float32),
        scratch_types=[
            pltpu.VMEM((_F, _BPW), jnp.int32),
            pltpu.VMEM((2, _BLK // 128, 128), jnp.int32),
            pltpu.VMEM((2, _BLK, 128), jnp.float32),
            pltpu.VMEM((2, _DIM, _BLK), jnp.float32),
        ] + [pltpu.SemaphoreType.DMA] * 4,
        mesh=plsc.VectorSubcoreMesh(core_axis_name="c", subcore_axis_name="s"),
        compiler_params=pltpu.CompilerParams(use_tc_tiling_on_sc=True, needs_layout_passes=False),
    )(_gather_body)

    sc4 = transpose(tab_t)
    out_t = gather(ind_t, sc4)
    return out_t.transpose(2, 0, 1)
